# baseline (speedup 1.0000x reference)
import functools

import numpy as np
import jax
import jax.numpy as jnp
from jax import lax
from jax.experimental import pallas as pl
from jax.experimental.pallas import tpu as pltpu
from jax.experimental.pallas import tpu_sc as plsc

_VOCAB = 1000000
_EMBED = 64
_BATCH = 4096
_SEQLEN = 200

_NC = 2
_NS = 16
_NW = _NC * _NS
_BBLK = _BATCH // _NW
_LANE = 16
_NBG = _BBLK // _LANE
_TI = _EMBED // 8
_KG = _EMBED // _LANE


def _pe_rows():
    pos = np.arange(_SEQLEN, dtype=np.float32)[:, None]
    div = np.exp(
        np.arange(0, _EMBED, 2, dtype=np.float32) * (-np.log(10000.0) / _EMBED)
    )
    pe = np.zeros((_SEQLEN, _EMBED), dtype=np.float32)
    pe[:, 0::2] = np.sin(pos * div)
    pe[:, 1::2] = np.cos(pos * div)
    return jnp.asarray(pe)


def _make_kernel():
    mesh = plsc.VectorSubcoreMesh(core_axis_name="c", subcore_axis_name="s")

    @functools.partial(
        pl.kernel,
        mesh=mesh,
        out_type=jax.ShapeDtypeStruct((_SEQLEN, _TI, _NW, 8, 128), jnp.float32),
        compiler_params=pltpu.CompilerParams(use_tc_tiling_on_sc=False,
                                             needs_layout_passes=False),
        scratch_types=(
            [pltpu.VMEM((_SEQLEN, _BBLK), jnp.int32)]
            + [pltpu.VMEM((_BBLK, _EMBED), jnp.float32)] * 4
            + [pltpu.VMEM((_EMBED,), jnp.float32)] * 4
            + [pltpu.VMEM((_TI, 8, 129), jnp.float32)] * 4
            + [pltpu.SemaphoreType.DMA] * 12
        ),
    )
    def emb_kernel(xT, pet_h, tblp, out5, xblk,
                   rows_0, rows_1, rows_2, rows_3,
                   pet_v0, pet_v1, pet_v2, pet_v3,
                   outt0, outt1, outt2, outt3,
                   gsem0, gsem1, gsem2, gsem3,
                   psem0, psem1, psem2, psem3,
                   osem0, osem1, osem2, osem3):
        wid = lax.axis_index("s") * _NC + lax.axis_index("c")
        b0 = wid * _BBLK

        rows = [rows_0, rows_1, rows_2, rows_3]
        pet_v = [pet_v0, pet_v1, pet_v2, pet_v3]
        outt = [outt0, outt1, outt2, outt3]
        gsem = [gsem0, gsem1, gsem2, gsem3]
        psem = [psem0, psem1, psem2, psem3]
        osem = [osem0, osem1, osem2, osem3]

        pltpu.sync_copy(xT.at[:, pl.ds(b0, _BBLK)], xblk)

        iota = lax.iota(jnp.int32, _LANE)

        def start_gather(l, buf):
            return pltpu.async_copy(tblp.at[xblk.at[l]], rows[buf], gsem[buf])

        def start_pet(l, buf):
            return pltpu.async_copy(pet_h.at[l], pet_v[buf], psem[buf])

        def wait_gather(buf):
            pltpu.make_async_copy(tblp.at[xblk.at[0]], rows[buf],
                                  gsem[buf]).wait()

        def wait_pet(buf):
            pltpu.make_async_copy(pet_h.at[0], pet_v[buf], psem[buf]).wait()

        def start_out(l, buf):
            return pltpu.async_copy(outt[buf].at[:, :, pl.ds(0, 128)],
                                    out5.at[l, :, wid], osem[buf])

        def wait_out(buf):
            pltpu.make_async_copy(outt[buf].at[:, :, pl.ds(0, 128)],
                                  out5.at[0, :, wid], osem[buf]).wait()

        tis = [(iota + k * _LANE) // 8 for k in range(_KG)]
        rvs = [lax.rem(iota + k * _LANE, 8) for k in range(_KG)]

        def compute(l, buf):
            pes = [pet_v[buf][pl.ds(k * _LANE, _LANE)] for k in range(_KG)]

            @plsc.parallel_loop(0, _BBLK, step=1, unroll=4)
            def j_body(j):
                jc = iota * 0 + j
                for k in range(_KG):
                    val = rows[buf][j, pl.ds(k * _LANE, _LANE)] + pes[k]
                    plsc.store_scatter(outt[buf], [tis[k], rvs[k], jc], val)

        for l0 in range(3):
            start_gather(l0, l0)
            start_pet(l0, l0)

        def outer(l2, carry):
            for b in range(4):
                l = l2 * 4 + b
                pfb = (b + 3) % 4

                def prefetch():
                    start_gather(l + 3, pfb)
                    start_pet(l + 3, pfb)
                if b == 0:
                    prefetch()
                else:
                    @pl.when(l2 < (_SEQLEN // 4) - 1)
                    def _():
                        prefetch()

                wait_gather(b)
                wait_pet(b)

                @pl.when(l2 > 0)
                def _():
                    wait_out(b)

                compute(l, b)
                start_out(l, b)
            return carry

        lax.fori_loop(0, _SEQLEN // 4, outer, 0)
        for b in range(4):
            wait_out(b)

    return emb_kernel


_EMB_KERNEL = _make_kernel()

_TBLK = 2048


def _tp_body(t_ref, o_ref):
    r = lax.broadcasted_iota(jnp.int32, (_EMBED, _EMBED), 0)
    c = lax.broadcasted_iota(jnp.int32, (_EMBED, _EMBED), 1)
    eye = (r == c).astype(jnp.float32)
    o_ref[:, : _EMBED] = lax.dot_general(
        t_ref[...], eye, (((0,), (0,)), ((), ())),
        preferred_element_type=jnp.float32)


def _pad_table(tT):
    n = (_VOCAB + _TBLK - 1) // _TBLK
    return pl.pallas_call(
        _tp_body,
        grid=(n,),
        in_specs=[pl.BlockSpec((_EMBED, _TBLK), lambda i: (0, i))],
        out_specs=pl.BlockSpec((_TBLK, 128), lambda i: (i, 0)),
        out_shape=jax.ShapeDtypeStruct((_VOCAB, 128), jnp.float32),
    )(tT)


def kernel(x, table):
    xT2 = x.T * 2
    tblp = _pad_table(table.T)
    tblh = tblp.reshape(2 * _VOCAB, _EMBED)
    out5 = _EMB_KERNEL(xT2, _pe_rows(), tblh)
    return out5.transpose(2, 4, 0, 1, 3).reshape(_BATCH, _SEQLEN, _EMBED)


# speedup vs baseline: 1.5017x; 1.5017x over previous
import functools

import numpy as np
import jax
import jax.numpy as jnp
from jax import lax
from jax.experimental import pallas as pl
from jax.experimental.pallas import tpu as pltpu
from jax.experimental.pallas import tpu_sc as plsc

_VOCAB = 1000000
_EMBED = 64
_BATCH = 4096
_SEQLEN = 200

_NC = 2
_NS = 16
_NW = _NC * _NS
_BBLK = _BATCH // _NW
_LANE = 16
_NBG = _BBLK // _LANE
_TI = _EMBED // 8
_KG = _EMBED // _LANE


def _pe_rows():
    pos = np.arange(_SEQLEN, dtype=np.float32)[:, None]
    div = np.exp(
        np.arange(0, _EMBED, 2, dtype=np.float32) * (-np.log(10000.0) / _EMBED)
    )
    pe = np.zeros((_SEQLEN, _EMBED), dtype=np.float32)
    pe[:, 0::2] = np.sin(pos * div)
    pe[:, 1::2] = np.cos(pos * div)
    return jnp.asarray(pe)


def _make_kernel():
    mesh = plsc.VectorSubcoreMesh(core_axis_name="c", subcore_axis_name="s")

    @functools.partial(
        pl.kernel,
        mesh=mesh,
        out_type=jax.ShapeDtypeStruct((_SEQLEN, _TI, _NW, 8, 128), jnp.float32),
        compiler_params=pltpu.CompilerParams(use_tc_tiling_on_sc=False,
                                             needs_layout_passes=False),
        scratch_types=(
            [pltpu.VMEM((_SEQLEN, _BBLK), jnp.int32)]
            + [pltpu.VMEM((_BBLK, _EMBED), jnp.float32)] * 4
            + [pltpu.VMEM((_EMBED,), jnp.float32)] * 4
            + [pltpu.VMEM((_TI, 8, 129), jnp.float32)] * 4
            + [pltpu.SemaphoreType.DMA] * 12
        ),
    )
    def emb_kernel(xT, pet_h, tblp, out5, xblk,
                   rows_0, rows_1, rows_2, rows_3,
                   pet_v0, pet_v1, pet_v2, pet_v3,
                   outt0, outt1, outt2, outt3,
                   gsem0, gsem1, gsem2, gsem3,
                   psem0, psem1, psem2, psem3,
                   osem0, osem1, osem2, osem3):
        wid = lax.axis_index("s") * _NC + lax.axis_index("c")
        b0 = wid * _BBLK

        rows = [rows_0, rows_1, rows_2, rows_3]
        pet_v = [pet_v0, pet_v1, pet_v2, pet_v3]
        outt = [outt0, outt1, outt2, outt3]
        gsem = [gsem0, gsem1, gsem2, gsem3]
        psem = [psem0, psem1, psem2, psem3]
        osem = [osem0, osem1, osem2, osem3]

        pltpu.sync_copy(xT.at[:, pl.ds(b0, _BBLK)], xblk)

        iota = lax.iota(jnp.int32, _LANE)

        def start_gather(l, buf):
            return pltpu.async_copy(tblp.at[xblk.at[l]], rows[buf], gsem[buf])

        def start_pet(l, buf):
            return pltpu.async_copy(pet_h.at[l], pet_v[buf], psem[buf])

        def wait_gather(buf):
            pltpu.make_async_copy(tblp.at[xblk.at[0]], rows[buf],
                                  gsem[buf]).wait()

        def wait_pet(buf):
            pltpu.make_async_copy(pet_h.at[0], pet_v[buf], psem[buf]).wait()

        def start_out(l, buf):
            return pltpu.async_copy(outt[buf].at[:, :, pl.ds(0, 128)],
                                    out5.at[l, :, wid], osem[buf])

        def wait_out(buf):
            pltpu.make_async_copy(outt[buf].at[:, :, pl.ds(0, 128)],
                                  out5.at[0, :, wid], osem[buf]).wait()

        tis = [(iota + k * _LANE) // 8 for k in range(_KG)]
        rvs = [lax.rem(iota + k * _LANE, 8) for k in range(_KG)]

        def compute(l, buf):
            pes = [pet_v[buf][pl.ds(k * _LANE, _LANE)] for k in range(_KG)]

            @plsc.parallel_loop(0, _BBLK, step=1, unroll=4)
            def j_body(j):
                jc = iota * 0 + j
                for k in range(_KG):
                    val = rows[buf][j, pl.ds(k * _LANE, _LANE)] + pes[k]
                    plsc.store_scatter(outt[buf], [tis[k], rvs[k], jc], val)

        for l0 in range(3):
            start_gather(l0, l0)
            start_pet(l0, l0)

        def outer(l2, carry):
            for b in range(4):
                l = l2 * 4 + b
                pfb = (b + 3) % 4

                def prefetch():
                    start_gather(l + 3, pfb)
                    start_pet(l + 3, pfb)
                if b == 0:
                    prefetch()
                else:
                    @pl.when(l2 < (_SEQLEN // 4) - 1)
                    def _():
                        prefetch()

                wait_gather(b)
                wait_pet(b)

                @pl.when(l2 > 0)
                def _():
                    wait_out(b)

                compute(l, b)
                start_out(l, b)
            return carry

        lax.fori_loop(0, _SEQLEN // 4, outer, 0)
        for b in range(4):
            wait_out(b)

    return emb_kernel


_EMB_KERNEL = _make_kernel()

_TBLK = 8192


def _tp_body(t_ref, o_ref):
    o_ref[:, : _EMBED] = t_ref[...].T


def _pad_table(tT):
    n = (_VOCAB + _TBLK - 1) // _TBLK
    return pl.pallas_call(
        _tp_body,
        grid=(n,),
        in_specs=[pl.BlockSpec((_EMBED, _TBLK), lambda i: (0, i))],
        out_specs=pl.BlockSpec((_TBLK, 128), lambda i: (i, 0)),
        out_shape=jax.ShapeDtypeStruct((_VOCAB, 128), jnp.float32),
    )(tT)


def kernel(x, table):
    xT2 = x.T * 2
    tblp = _pad_table(table.T)
    tblh = tblp.reshape(2 * _VOCAB, _EMBED)
    out5 = _EMB_KERNEL(xT2, _pe_rows(), tblh)
    return out5.transpose(2, 4, 0, 1, 3).reshape(_BATCH, _SEQLEN, _EMBED)


# speedup vs baseline: 1.5750x; 1.0488x over previous
import functools

import numpy as np
import jax
import jax.numpy as jnp
from jax import lax
from jax.experimental import pallas as pl
from jax.experimental.pallas import tpu as pltpu
from jax.experimental.pallas import tpu_sc as plsc

_VOCAB = 1000000
_EMBED = 64
_BATCH = 4096
_SEQLEN = 200

_NC = 2
_NS = 16
_NW = _NC * _NS
_BBLK = _BATCH // _NW
_LANE = 16
_NBG = _BBLK // _LANE
_TI = _EMBED // 8
_KG = _EMBED // _LANE


def _pe_rows():
    pos = np.arange(_SEQLEN, dtype=np.float32)[:, None]
    div = np.exp(
        np.arange(0, _EMBED, 2, dtype=np.float32) * (-np.log(10000.0) / _EMBED)
    )
    pe = np.zeros((_SEQLEN, _EMBED), dtype=np.float32)
    pe[:, 0::2] = np.sin(pos * div)
    pe[:, 1::2] = np.cos(pos * div)
    return jnp.asarray(pe)


def _make_kernel():
    mesh = plsc.VectorSubcoreMesh(core_axis_name="c", subcore_axis_name="s")

    @functools.partial(
        pl.kernel,
        mesh=mesh,
        out_type=jax.ShapeDtypeStruct((_SEQLEN, _TI, _NW, 8, 128), jnp.float32),
        compiler_params=pltpu.CompilerParams(use_tc_tiling_on_sc=False,
                                             needs_layout_passes=False),
        scratch_types=(
            [pltpu.VMEM((_SEQLEN, _BBLK), jnp.int32)]
            + [pltpu.VMEM((_BBLK, _EMBED), jnp.float32)] * 4
            + [pltpu.VMEM((_EMBED,), jnp.float32)] * 4
            + [pltpu.VMEM((_TI, 8, 129), jnp.float32)] * 4
            + [pltpu.SemaphoreType.DMA] * 12
        ),
    )
    def emb_kernel(xT, pet_h, tblp, out5, xblk,
                   rows_0, rows_1, rows_2, rows_3,
                   pet_v0, pet_v1, pet_v2, pet_v3,
                   outt0, outt1, outt2, outt3,
                   gsem0, gsem1, gsem2, gsem3,
                   psem0, psem1, psem2, psem3,
                   osem0, osem1, osem2, osem3):
        wid = lax.axis_index("s") * _NC + lax.axis_index("c")
        b0 = wid * _BBLK

        rows = [rows_0, rows_1, rows_2, rows_3]
        pet_v = [pet_v0, pet_v1, pet_v2, pet_v3]
        outt = [outt0, outt1, outt2, outt3]
        gsem = [gsem0, gsem1, gsem2, gsem3]
        psem = [psem0, psem1, psem2, psem3]
        osem = [osem0, osem1, osem2, osem3]

        pltpu.sync_copy(xT.at[:, pl.ds(b0, _BBLK)], xblk)

        iota = lax.iota(jnp.int32, _LANE)

        def start_gather(l, buf):
            return pltpu.async_copy(tblp.at[xblk.at[l]], rows[buf], gsem[buf])

        def start_pet(l, buf):
            return pltpu.async_copy(pet_h.at[l], pet_v[buf], psem[buf])

        def wait_gather(buf):
            pltpu.make_async_copy(tblp.at[xblk.at[0]], rows[buf],
                                  gsem[buf]).wait()

        def wait_pet(buf):
            pltpu.make_async_copy(pet_h.at[0], pet_v[buf], psem[buf]).wait()

        def start_out(l, buf):
            return pltpu.async_copy(outt[buf].at[:, :, pl.ds(0, 128)],
                                    out5.at[l, :, wid], osem[buf])

        def wait_out(buf):
            pltpu.make_async_copy(outt[buf].at[:, :, pl.ds(0, 128)],
                                  out5.at[0, :, wid], osem[buf]).wait()

        tis = [(iota + k * _LANE) // 8 for k in range(_KG)]
        rvs = [lax.rem(iota + k * _LANE, 8) for k in range(_KG)]

        def compute(l, buf):
            pes = [pet_v[buf][pl.ds(k * _LANE, _LANE)] for k in range(_KG)]

            @plsc.parallel_loop(0, _BBLK, step=1, unroll=4)
            def j_body(j):
                jc = iota * 0 + j
                for k in range(_KG):
                    val = rows[buf][j, pl.ds(k * _LANE, _LANE)] + pes[k]
                    plsc.store_scatter(outt[buf], [tis[k], rvs[k], jc], val)

        for l0 in range(3):
            start_gather(l0, l0)
            start_pet(l0, l0)

        def outer(l2, carry):
            for b in range(4):
                l = l2 * 4 + b
                pfb = (b + 3) % 4

                def prefetch():
                    start_gather(l + 3, pfb)
                    start_pet(l + 3, pfb)
                if b == 0:
                    prefetch()
                else:
                    @pl.when(l2 < (_SEQLEN // 4) - 1)
                    def _():
                        prefetch()

                wait_gather(b)
                wait_pet(b)

                @pl.when(l2 > 0)
                def _():
                    wait_out(b)

                compute(l, b)
                start_out(l, b)
            return carry

        lax.fori_loop(0, _SEQLEN // 4, outer, 0)
        for b in range(4):
            wait_out(b)

    return emb_kernel


_EMB_KERNEL = _make_kernel()

_TBLK = 16384


def _tp_body(t_ref, o_ref):
    o_ref[:, : _EMBED] = t_ref[...].T


def _pad_table(tT):
    n = (_VOCAB + _TBLK - 1) // _TBLK
    return pl.pallas_call(
        _tp_body,
        grid=(n,),
        in_specs=[pl.BlockSpec((_EMBED, _TBLK), lambda i: (0, i))],
        out_specs=pl.BlockSpec((_TBLK, 128), lambda i: (i, 0)),
        out_shape=jax.ShapeDtypeStruct((_VOCAB, 128), jnp.float32),
    )(tT)


def kernel(x, table):
    xT2 = x.T * 2
    tblp = _pad_table(table.T)
    tblh = tblp.reshape(2 * _VOCAB, _EMBED)
    out5 = _EMB_KERNEL(xT2, _pe_rows(), tblh)
    return out5.transpose(2, 4, 0, 1, 3).reshape(_BATCH, _SEQLEN, _EMBED)


# speedup vs baseline: 1.6021x; 1.0172x over previous
import functools

import numpy as np
import jax
import jax.numpy as jnp
from jax import lax
from jax.experimental import pallas as pl
from jax.experimental.pallas import tpu as pltpu
from jax.experimental.pallas import tpu_sc as plsc

_VOCAB = 1000000
_EMBED = 64
_BATCH = 4096
_SEQLEN = 200

_NC = 2
_NS = 16
_NW = _NC * _NS
_BBLK = _BATCH // _NW
_LANE = 16
_NBG = _BBLK // _LANE
_TI = _EMBED // 8
_KG = _EMBED // _LANE


def _pe_rows():
    pos = np.arange(_SEQLEN, dtype=np.float32)[:, None]
    div = np.exp(
        np.arange(0, _EMBED, 2, dtype=np.float32) * (-np.log(10000.0) / _EMBED)
    )
    pe = np.zeros((_SEQLEN, _EMBED), dtype=np.float32)
    pe[:, 0::2] = np.sin(pos * div)
    pe[:, 1::2] = np.cos(pos * div)
    return jnp.asarray(pe)


def _make_kernel():
    mesh = plsc.VectorSubcoreMesh(core_axis_name="c", subcore_axis_name="s")

    @functools.partial(
        pl.kernel,
        mesh=mesh,
        out_type=jax.ShapeDtypeStruct((_SEQLEN, _TI, _NW, 8, 128), jnp.float32),
        compiler_params=pltpu.CompilerParams(use_tc_tiling_on_sc=False,
                                             needs_layout_passes=False),
        scratch_types=(
            [pltpu.VMEM((_SEQLEN, _BBLK), jnp.int32)]
            + [pltpu.VMEM((_BBLK, _EMBED), jnp.float32)] * 4
            + [pltpu.VMEM((_EMBED,), jnp.float32)] * 4
            + [pltpu.VMEM((_TI, 8, 129), jnp.float32)] * 4
            + [pltpu.SemaphoreType.DMA] * 12
        ),
    )
    def emb_kernel(xT, pet_h, tblp, out5, xblk,
                   rows_0, rows_1, rows_2, rows_3,
                   pet_v0, pet_v1, pet_v2, pet_v3,
                   outt0, outt1, outt2, outt3,
                   gsem0, gsem1, gsem2, gsem3,
                   psem0, psem1, psem2, psem3,
                   osem0, osem1, osem2, osem3):
        wid = lax.axis_index("s") * _NC + lax.axis_index("c")
        b0 = wid * _BBLK

        rows = [rows_0, rows_1, rows_2, rows_3]
        pet_v = [pet_v0, pet_v1, pet_v2, pet_v3]
        outt = [outt0, outt1, outt2, outt3]
        gsem = [gsem0, gsem1, gsem2, gsem3]
        psem = [psem0, psem1, psem2, psem3]
        osem = [osem0, osem1, osem2, osem3]

        pltpu.sync_copy(xT.at[:, pl.ds(b0, _BBLK)], xblk)

        iota = lax.iota(jnp.int32, _LANE)

        def start_gather(l, buf):
            return pltpu.async_copy(tblp.at[xblk.at[l]], rows[buf], gsem[buf])

        def start_pet(l, buf):
            return pltpu.async_copy(pet_h.at[l], pet_v[buf], psem[buf])

        def wait_gather(buf):
            pltpu.make_async_copy(tblp.at[xblk.at[0]], rows[buf],
                                  gsem[buf]).wait()

        def wait_pet(buf):
            pltpu.make_async_copy(pet_h.at[0], pet_v[buf], psem[buf]).wait()

        def start_out(l, buf):
            return pltpu.async_copy(outt[buf].at[:, :, pl.ds(0, 128)],
                                    out5.at[l, :, wid], osem[buf])

        def wait_out(buf):
            pltpu.make_async_copy(outt[buf].at[:, :, pl.ds(0, 128)],
                                  out5.at[0, :, wid], osem[buf]).wait()

        tis = [(iota + k * _LANE) // 8 for k in range(_KG)]
        rvs = [lax.rem(iota + k * _LANE, 8) for k in range(_KG)]

        def compute(l, buf):
            pes = [pet_v[buf][pl.ds(k * _LANE, _LANE)] for k in range(_KG)]

            @plsc.parallel_loop(0, _BBLK, step=1, unroll=4)
            def j_body(j):
                jc = iota * 0 + j
                for k in range(_KG):
                    val = rows[buf][j, pl.ds(k * _LANE, _LANE)] + pes[k]
                    plsc.store_scatter(outt[buf], [tis[k], rvs[k], jc], val)

        for l0 in range(3):
            start_gather(l0, l0)
            start_pet(l0, l0)

        def outer(l2, carry):
            for b in range(4):
                l = l2 * 4 + b
                pfb = (b + 3) % 4

                def prefetch():
                    start_gather(l + 3, pfb)
                    start_pet(l + 3, pfb)
                if b == 0:
                    prefetch()
                else:
                    @pl.when(l2 < (_SEQLEN // 4) - 1)
                    def _():
                        prefetch()

                wait_gather(b)
                wait_pet(b)

                @pl.when(l2 > 0)
                def _():
                    wait_out(b)

                compute(l, b)
                start_out(l, b)
            return carry

        lax.fori_loop(0, _SEQLEN // 4, outer, 0)
        for b in range(4):
            wait_out(b)

    return emb_kernel


_EMB_KERNEL = _make_kernel()

_TBLK = 32768


def _tp_body(t_ref, o_ref):
    o_ref[:, : _EMBED] = t_ref[...].T


def _pad_table(tT):
    n = (_VOCAB + _TBLK - 1) // _TBLK
    return pl.pallas_call(
        _tp_body,
        grid=(n,),
        in_specs=[pl.BlockSpec((_EMBED, _TBLK), lambda i: (0, i))],
        out_specs=pl.BlockSpec((_TBLK, 128), lambda i: (i, 0)),
        out_shape=jax.ShapeDtypeStruct((_VOCAB, 128), jnp.float32),
    )(tT)


def kernel(x, table):
    xT2 = x.T * 2
    tblp = _pad_table(table.T)
    tblh = tblp.reshape(2 * _VOCAB, _EMBED)
    out5 = _EMB_KERNEL(xT2, _pe_rows(), tblh)
    return out5.transpose(2, 4, 0, 1, 3).reshape(_BATCH, _SEQLEN, _EMBED)
